# baseline (speedup 1.0000x reference)
import functools

import jax
import jax.numpy as jnp
from jax import lax
from jax.experimental import pallas as pl
from jax.experimental.pallas import tpu as pltpu
from jax.experimental.pallas import tpu_sc as plsc

N = 10000
E = 320000
D = 128
NSUB = 16
NP = N + 112
ROWS_PER_SUB = NP // NSUB
CHUNK = 128
NBUF = 2
GRP = 16
NCHUNK = 160
EDGES_PER_SUB = CHUNK * NCHUNK
E_PAD = EDGES_PER_SUB * NSUB


def _sc_accumulate(lbls, src, dst, zeros):
    mesh = plsc.VectorSubcoreMesh(core_axis_name="c", subcore_axis_name="s")

    @functools.partial(
        pl.kernel,
        out_type=(
            jax.ShapeDtypeStruct((2, NP, D), jnp.float32),
            jax.ShapeDtypeStruct((2 * NP,), jnp.float32),
        ),
        mesh=mesh,
        scratch_types=[
            pltpu.VMEM_SHARED((NP, D), jnp.float32),
            pltpu.VMEM_SHARED((NP,), jnp.float32),
            pltpu.VMEM((CHUNK,), jnp.float32),
            pltpu.VMEM((640,), jnp.float32),
            pltpu.VMEM((CHUNK,), jnp.int32),
            pltpu.VMEM((CHUNK,), jnp.int32),
            [pltpu.VMEM((CHUNK, D), jnp.float32) for _ in range(NBUF)],
            [pltpu.SemaphoreType.DMA for _ in range(NBUF)],
            [pltpu.SemaphoreType.DMA for _ in range(NBUF)],
            [pltpu.SemaphoreType.DMA for _ in range(NBUF)],
            pltpu.SemaphoreType.DMA,
        ],
    )
    def body(lbl_hbm, srcf_hbm, dstf_hbm, z_hbm, sum_hbm, cnt_hbm,
             acc, cnt_sh, ones, stage, sidx, didx, rows, gsems, ssems,
             csems, isem):
        c = lax.axis_index("c")
        s = lax.axis_index("s")
        r0 = s * ROWS_PER_SUB
        row0 = (c * NSUB + s) * NCHUNK
        pltpu.sync_copy(z_hbm, acc.at[pl.ds(r0, ROWS_PER_SUB)])
        ones16 = jnp.full((16,), 1.0, jnp.float32)
        for k in range(CHUNK // 16):
            ones[pl.ds(k * 16, 16)] = ones16
        zero16 = jnp.zeros((16,), jnp.float32)
        for k in range(640 // 16):
            stage[pl.ds(k * 16, 16)] = zero16
        pltpu.sync_copy(stage.at[pl.ds(0, ROWS_PER_SUB)],
                        cnt_sh.at[pl.ds(r0, ROWS_PER_SUB)])
        plsc.subcore_barrier()

        def step(j, carry):
            base = row0 * CHUNK + j * CHUNK
            pltpu.sync_copy(srcf_hbm.at[pl.ds(base, CHUNK)], sidx)
            pltpu.sync_copy(dstf_hbm.at[pl.ds(base, CHUNK)], didx)
            pltpu.async_copy(lbl_hbm.at[sidx], rows[0], gsems[0]).wait()
            s0 = pltpu.async_copy(
                rows[0], acc.at[didx], ssems[0], add=True)
            c0 = pltpu.async_copy(
                ones, cnt_sh.at[didx], csems[0], add=True)
            s0.wait()
            c0.wait()
            return carry

        lax.fori_loop(0, NCHUNK, step, 0)
        plsc.subcore_barrier()
        pltpu.sync_copy(acc.at[pl.ds(r0, ROWS_PER_SUB)],
                        sum_hbm.at[c, pl.ds(r0, ROWS_PER_SUB)])
        pltpu.sync_copy(cnt_sh.at[pl.ds(r0, ROWS_PER_SUB)],
                        stage.at[pl.ds(0, ROWS_PER_SUB)])
        pltpu.sync_copy(stage.at[pl.ds(0, ROWS_PER_SUB)],
                        cnt_hbm.at[pl.ds(c * NP + r0, ROWS_PER_SUB)])

    return body(lbls, src, dst, zeros)


def _tc_combine(sums_ref, cnts_ref, lbls_ref, msk_ref, out_ref):
    c1 = jnp.maximum(cnts_ref[0][:N, :], 1.0)
    c2 = jnp.maximum(cnts_ref[1][:N, :], 1.0)
    m = 0.5 * (sums_ref[0][:N, :] / c1 + sums_ref[1][:N, :] / c2)
    out_ref[...] = jnp.where(msk_ref[...] > 0, m, lbls_ref[...])


def kernel(lbls, no_lbl_idx, knn_sc, knn_fc):
    pad = E_PAD - E
    pad_dst = (jnp.arange(pad, dtype=jnp.int32) % (NP - N)) + N
    zpad = jnp.zeros((pad,), jnp.int32)
    src = jnp.concatenate([knn_sc[0], zpad, knn_fc[0], zpad])
    dst = jnp.concatenate([knn_sc[1], pad_dst, knn_fc[1], pad_dst])
    zeros = jnp.zeros((ROWS_PER_SUB, D), jnp.float32)

    sums, cnts = _sc_accumulate(lbls, src, dst, zeros)

    msk = no_lbl_idx.astype(jnp.int32).reshape(N, 1)
    return pl.pallas_call(
        _tc_combine,
        out_shape=jax.ShapeDtypeStruct((N, D), jnp.float32),
    )(sums, cnts.reshape(2, NP, 1), lbls, msk)


# speedup vs baseline: 1.6476x; 1.6476x over previous
import functools

import jax
import jax.numpy as jnp
from jax import lax
from jax.experimental import pallas as pl
from jax.experimental.pallas import tpu as pltpu
from jax.experimental.pallas import tpu_sc as plsc

N = 10000
E = 320000
D = 128
NSUB = 16
NP = N + 112
ROWS_PER_SUB = NP // NSUB
CHUNK = 128
NCHUNK = 157
EDGES_PER_SUB = CHUNK * NCHUNK
E_PAD = EDGES_PER_SUB * NSUB


def _sc_accumulate(table, src, dst, zeros2d):
    mesh = plsc.VectorSubcoreMesh(core_axis_name="c", subcore_axis_name="s")

    @functools.partial(
        pl.kernel,
        out_type=(
            jax.ShapeDtypeStruct((2, NP, D), jnp.float32),
            jax.ShapeDtypeStruct((2 * NP,), jnp.float32),
        ),
        mesh=mesh,
        scratch_types=[
            pltpu.VMEM_SHARED((NP, D), jnp.float32),
            pltpu.VMEM_SHARED((NP,), jnp.float32),
            pltpu.VMEM((CHUNK,), jnp.float32),
            pltpu.VMEM((640,), jnp.float32),
            pltpu.VMEM((CHUNK,), jnp.int32),
            pltpu.VMEM((CHUNK,), jnp.int32),
            pltpu.VMEM((CHUNK, D), jnp.float32),
            pltpu.SemaphoreType.DMA,
        ],
    )
    def body(tab_hbm, src_hbm, dst_hbm, z2_hbm, sum_hbm, cnt_hbm,
             acc, cnt_sh, ones, stage, sidx, didx, rows, sem):
        c = lax.axis_index("c")
        s = lax.axis_index("s")
        r0 = s * ROWS_PER_SUB
        pltpu.sync_copy(z2_hbm.at[pl.ds(r0, ROWS_PER_SUB)],
                        acc.at[pl.ds(r0, ROWS_PER_SUB)])
        ones16 = jnp.full((16,), 1.0, jnp.float32)
        for k in range(CHUNK // 16):
            ones[pl.ds(k * 16, 16)] = ones16
        zero16 = jnp.zeros((16,), jnp.float32)
        for k in range(640 // 16):
            stage[pl.ds(k * 16, 16)] = zero16
        pltpu.sync_copy(stage.at[pl.ds(0, ROWS_PER_SUB)],
                        cnt_sh.at[pl.ds(r0, ROWS_PER_SUB)])
        plsc.subcore_barrier()

        base0 = c * E_PAD + s * EDGES_PER_SUB

        def step(j, carry):
            base = base0 + j * CHUNK
            pltpu.sync_copy(src_hbm.at[pl.ds(base, CHUNK)], sidx)
            pltpu.sync_copy(dst_hbm.at[pl.ds(base, CHUNK)], didx)
            pltpu.async_copy(tab_hbm.at[sidx], rows, sem).wait()
            pltpu.sync_copy(rows, acc.at[didx], add=True)
            pltpu.sync_copy(ones, cnt_sh.at[didx], add=True)
            return carry

        lax.fori_loop(0, NCHUNK, step, 0)
        plsc.subcore_barrier()
        pltpu.sync_copy(acc.at[pl.ds(r0, ROWS_PER_SUB)],
                        sum_hbm.at[c, pl.ds(r0, ROWS_PER_SUB)])
        pltpu.sync_copy(cnt_sh.at[pl.ds(r0, ROWS_PER_SUB)],
                        stage.at[pl.ds(0, ROWS_PER_SUB)])
        pltpu.sync_copy(stage.at[pl.ds(0, ROWS_PER_SUB)],
                        cnt_hbm.at[pl.ds(c * NP + r0, ROWS_PER_SUB)])

    return body(table, src, dst, zeros2d)


def _tc_combine(sums_ref, cnts_ref, lbls_ref, msk_ref, out_ref):
    c1 = jnp.maximum(cnts_ref[0][:N, :], 1.0)
    c2 = jnp.maximum(cnts_ref[1][:N, :], 1.0)
    m = 0.5 * (sums_ref[0][:N, :] / c1 + sums_ref[1][:N, :] / c2)
    out_ref[...] = jnp.where(msk_ref[...] > 0, m, lbls_ref[...])


def kernel(lbls, no_lbl_idx, knn_sc, knn_fc):
    table = jnp.zeros((NP, D), jnp.float32).at[:N].set(lbls)
    pad = ((0, 0), (0, E_PAD - E))
    src = jnp.pad(jnp.stack([knn_sc[0], knn_fc[0]]), pad,
                  constant_values=N).reshape(2 * E_PAD)
    dst = jnp.pad(jnp.stack([knn_sc[1], knn_fc[1]]), pad,
                  constant_values=N).reshape(2 * E_PAD)
    zeros2d = jnp.zeros((NP, D), jnp.float32)

    sums, cnts = _sc_accumulate(table, src, dst, zeros2d)

    msk = no_lbl_idx.astype(jnp.int32).reshape(N, 1)
    return pl.pallas_call(
        _tc_combine,
        out_shape=jax.ShapeDtypeStruct((N, D), jnp.float32),
    )(sums, cnts.reshape(2, NP, 1), lbls, msk)


# speedup vs baseline: 1.6483x; 1.0005x over previous
import functools

import jax
import jax.numpy as jnp
from jax import lax
from jax.experimental import pallas as pl
from jax.experimental.pallas import tpu as pltpu
from jax.experimental.pallas import tpu_sc as plsc

N = 10000
E = 320000
D = 128
NSUB = 16
NP = N + 112
ROWS_PER_SUB = NP // NSUB
CHUNK = 128
NCHUNK = 157
EDGES_PER_SUB = CHUNK * NCHUNK
E_PAD = EDGES_PER_SUB * NSUB


def _sc_accumulate(table, src, dst, zeros2d):
    mesh = plsc.VectorSubcoreMesh(core_axis_name="c", subcore_axis_name="s")

    @functools.partial(
        pl.kernel,
        out_type=(
            jax.ShapeDtypeStruct((2, NP, D), jnp.float32),
            jax.ShapeDtypeStruct((2 * NP,), jnp.float32),
        ),
        mesh=mesh,
        scratch_types=[
            pltpu.VMEM_SHARED((NP, D), jnp.float32),
            pltpu.VMEM_SHARED((NP,), jnp.float32),
            pltpu.VMEM((CHUNK,), jnp.float32),
            pltpu.VMEM((640,), jnp.float32),
            pltpu.VMEM((CHUNK,), jnp.int32),
            pltpu.VMEM((CHUNK,), jnp.int32),
            pltpu.VMEM((CHUNK, D), jnp.float32),
            pltpu.SemaphoreType.DMA,
        ],
    )
    def body(tab_hbm, src_hbm, dst_hbm, z2_hbm, sum_hbm, cnt_hbm,
             acc, cnt_sh, ones, stage, sidx, didx, rows, sem):
        c = lax.axis_index("c")
        s = lax.axis_index("s")
        r0 = s * ROWS_PER_SUB
        pltpu.sync_copy(z2_hbm.at[pl.ds(r0, ROWS_PER_SUB)],
                        acc.at[pl.ds(r0, ROWS_PER_SUB)])
        ones16 = jnp.full((16,), 1.0, jnp.float32)
        for k in range(CHUNK // 16):
            ones[pl.ds(k * 16, 16)] = ones16
        zero16 = jnp.zeros((16,), jnp.float32)
        for k in range(640 // 16):
            stage[pl.ds(k * 16, 16)] = zero16
        pltpu.sync_copy(stage.at[pl.ds(0, ROWS_PER_SUB)],
                        cnt_sh.at[pl.ds(r0, ROWS_PER_SUB)])
        plsc.subcore_barrier()

        base0 = c * E_PAD + s * EDGES_PER_SUB

        def step(j, carry):
            base = base0 + j * CHUNK
            pltpu.sync_copy(src_hbm.at[pl.ds(base, CHUNK)], sidx)
            pltpu.sync_copy(dst_hbm.at[pl.ds(base, CHUNK)], didx)
            pltpu.sync_copy(tab_hbm.at[sidx], rows)
            pltpu.sync_copy(rows, acc.at[didx], add=True)
            pltpu.sync_copy(ones, cnt_sh.at[didx], add=True)
            return carry

        lax.fori_loop(0, NCHUNK, step, 0)
        plsc.subcore_barrier()
        pltpu.sync_copy(acc.at[pl.ds(r0, ROWS_PER_SUB)],
                        sum_hbm.at[c, pl.ds(r0, ROWS_PER_SUB)])
        pltpu.sync_copy(cnt_sh.at[pl.ds(r0, ROWS_PER_SUB)],
                        stage.at[pl.ds(0, ROWS_PER_SUB)])
        pltpu.sync_copy(stage.at[pl.ds(0, ROWS_PER_SUB)],
                        cnt_hbm.at[pl.ds(c * NP + r0, ROWS_PER_SUB)])

    return body(table, src, dst, zeros2d)


def _tc_combine(sums_ref, cnts_ref, lbls_ref, msk_ref, out_ref):
    c1 = jnp.maximum(cnts_ref[0][:N, :], 1.0)
    c2 = jnp.maximum(cnts_ref[1][:N, :], 1.0)
    m = 0.5 * (sums_ref[0][:N, :] / c1 + sums_ref[1][:N, :] / c2)
    out_ref[...] = jnp.where(msk_ref[...] > 0, m, lbls_ref[...])


def kernel(lbls, no_lbl_idx, knn_sc, knn_fc):
    table = jnp.zeros((NP, D), jnp.float32).at[:N].set(lbls)
    pad = ((0, 0), (0, E_PAD - E))
    src = jnp.pad(jnp.stack([knn_sc[0], knn_fc[0]]), pad,
                  constant_values=N).reshape(2 * E_PAD)
    dst = jnp.pad(jnp.stack([knn_sc[1], knn_fc[1]]), pad,
                  constant_values=N).reshape(2 * E_PAD)
    zeros2d = jnp.zeros((NP, D), jnp.float32)

    sums, cnts = _sc_accumulate(table, src, dst, zeros2d)

    msk = no_lbl_idx.astype(jnp.int32).reshape(N, 1)
    return pl.pallas_call(
        _tc_combine,
        out_shape=jax.ShapeDtypeStruct((N, D), jnp.float32),
    )(sums, cnts.reshape(2, NP, 1), lbls, msk)
